# baseline (speedup 1.0000x reference)
import functools

import jax
import jax.numpy as jnp
from jax import lax
from jax.experimental import pallas as pl
from jax.experimental.pallas import tpu as pltpu
from jax.experimental.pallas import tpu_sc as plsc

_N = 10000
_E = 320000
_F = 128
_NCLASS = 40

_NC = 2
_NS = 16
_NW = _NC * _NS
_C = 64
_TB = 16
_K = 4
_T0 = 320
_T1 = 0
_EPAD = _NS * (_T0 + _T1) * _C
_RPT = 632
_NACC = _RPT * _NS

_BR = 1000


def _make_agg_kernel():
    mesh = plsc.VectorSubcoreMesh(core_axis_name="c", subcore_axis_name="s")

    @functools.partial(
        pl.kernel,
        out_type=jax.ShapeDtypeStruct((_NC, _NACC, _F), jnp.float32),
        mesh=mesh,
        scratch_types=[
            pltpu.VMEM((_TB, _C), jnp.int32),
            pltpu.VMEM((_TB, _C), jnp.int32),
            [pltpu.VMEM((_C, _F), jnp.float32) for _ in range(_K)],
            pltpu.VMEM_SHARED((_NACC, _F), jnp.float32),
            [pltpu.SemaphoreType.DMA for _ in range(_K)],
            [pltpu.SemaphoreType.DMA for _ in range(_K)],
        ],
    )
    def agg(h_hbm, srcb_hbm, dstb_hbm,
            out_hbm, src_v, dst_v, bufs, acc_sh, gs, ps):
        c = lax.axis_index("c")
        s = lax.axis_index("s")

        @pl.when((c == 0) | (_T1 > 0))
        def _():
            zvec = jnp.zeros((16,), jnp.float32)

            def zrow(r, carry):
                for j2 in range(_F // 16):
                    bufs[0][r, pl.ds(j2 * 16, 16)] = zvec
                return carry

            lax.fori_loop(0, _C, zrow, 0)

            nfull = _RPT // _C
            rem = _RPT - nfull * _C

            def crow(k, carry):
                pltpu.sync_copy(bufs[0],
                                acc_sh.at[pl.ds(s * _RPT + k * _C, _C)])
                return carry

            lax.fori_loop(0, nfull, crow, 0)
            if rem:
                pltpu.sync_copy(
                    bufs[0].at[pl.ds(0, rem)],
                    acc_sh.at[pl.ds(s * _RPT + nfull * _C, rem)])
        plsc.subcore_barrier()

        def start_gather(t, j):
            pltpu.async_copy(h_hbm.at[src_v.at[t]], bufs[j], gs[j])

        def wait_gather(j):
            pltpu.make_async_copy(h_hbm.at[src_v.at[0]], bufs[j],
                                  gs[j]).wait()

        def start_scatter(t, j):
            pltpu.async_copy(bufs[j], acc_sh.at[dst_v.at[t]], ps[j],
                             add=True)

        def wait_scatter(j):
            pltpu.make_async_copy(bufs[j], acc_sh.at[dst_v.at[0]],
                                  ps[j]).wait()

        nb0 = _T0 // _TB
        nb1 = _T1 // _TB
        nblk = jnp.where(c == 0, nb0, nb1)
        bbase = jnp.where(c == 0, s * nb0, _NS * nb0 + s * nb1)

        def blk_body(blk, carry):
            pltpu.sync_copy(srcb_hbm.at[bbase + blk], src_v)
            pltpu.sync_copy(dstb_hbm.at[bbase + blk], dst_v)
            for j in range(_K - 1):
                start_gather(j, j)

            for j in range(_K):
                wait_gather(j)
                start_scatter(j, j)
                jn = (j + _K - 1) % _K
                if j > 0:
                    wait_scatter(jn)
                start_gather(j + _K - 1, jn)

            def group_body(g, carry2):
                base = g * _K
                for j in range(_K):
                    t = base + j
                    wait_gather(j)
                    start_scatter(t, j)
                    jn = (j + _K - 1) % _K
                    wait_scatter(jn)
                    start_gather(t + _K - 1, jn)
                return carry2

            lax.fori_loop(1, _TB // _K - 1, group_body, 0)

            base = _TB - _K
            wait_gather(0)
            start_scatter(base, 0)
            wait_scatter(_K - 1)
            start_gather(_TB - 1, _K - 1)
            for j in range(1, _K):
                wait_gather(j)
                start_scatter(base + j, j)
            for j in range(_K):
                wait_scatter(j)
            return carry

        lax.fori_loop(0, nblk, blk_body, 0)

        plsc.subcore_barrier()

        @pl.when((c == 0) | (_T1 > 0))
        def _():
            pltpu.sync_copy(acc_sh.at[pl.ds(s * _RPT, _RPT)],
                            out_hbm.at[c].at[pl.ds(s * _RPT, _RPT)])

    return agg


_agg = _make_agg_kernel()


def _mm_relu_body(x_ref, w_ref, b_ref, o_ref):
    z = jnp.dot(x_ref[...], w_ref[...], preferred_element_type=jnp.float32)
    o_ref[...] = jnp.maximum(z + b_ref[...], 0.0)


_NAGG = 2 if _T1 > 0 else 1


def _gin_mm_relu_body(h_ref, *refs):
    aggs, (w_ref, b_ref, o_ref) = refs[:_NAGG], refs[_NAGG:]
    z = h_ref[...]
    for a_ref in aggs:
        z = z + a_ref[...]
    z = jnp.dot(z, w_ref[...], preferred_element_type=jnp.float32)
    o_ref[...] = jnp.maximum(z + b_ref[...], 0.0)


def _final_body(h_ref, *refs):
    aggs, (w_ref, b_ref, fw_ref, fb_ref, o_ref) = refs[:_NAGG], refs[_NAGG:]
    z = h_ref[...]
    for a_ref in aggs:
        z = z + a_ref[...]
    z = jnp.dot(z, w_ref[...], preferred_element_type=jnp.float32)
    h2 = jnp.maximum(z + b_ref[...], 0.0)
    logits = jnp.dot(h2, fw_ref[...], preferred_element_type=jnp.float32)
    logits = logits + fb_ref[...]
    col = lax.broadcasted_iota(jnp.int32, logits.shape, 1)
    logits = jnp.where(col < _NCLASS, logits, -1e30)
    m = jnp.max(logits, axis=1, keepdims=True)
    lse = jnp.log(jnp.sum(jnp.exp(logits - m), axis=1, keepdims=True)) + m
    o_ref[...] = logits - lse


def _row_spec():
    return pl.BlockSpec((_BR, _F), lambda i: (i, 0))


def _full_spec(shape):
    return pl.BlockSpec(shape, lambda i: (0,) * len(shape))


def _grid():
    return (_N + _BR - 1) // _BR


def _mm_relu(x, w, b):
    return pl.pallas_call(
        _mm_relu_body,
        grid=(_grid(),),
        in_specs=[_row_spec(), _full_spec((_F, _F)), _full_spec((1, _F))],
        out_specs=_row_spec(),
        out_shape=jax.ShapeDtypeStruct((_N, _F), jnp.float32),
    )(x, w, b)


def _gin_mm_relu(h, aggs, w, b):
    return pl.pallas_call(
        _gin_mm_relu_body,
        grid=(_grid(),),
        in_specs=[_row_spec()] * (1 + _NAGG)
        + [_full_spec((_F, _F)), _full_spec((1, _F))],
        out_specs=_row_spec(),
        out_shape=jax.ShapeDtypeStruct((_N, _F), jnp.float32),
    )(h, *aggs, w, b)


def _final(h, aggs, w, b, fw, fb):
    return pl.pallas_call(
        _final_body,
        grid=(_grid(),),
        in_specs=[_row_spec()] * (1 + _NAGG)
        + [_full_spec((_F, _F)), _full_spec((1, _F)),
           _full_spec((_F, _F)), _full_spec((1, _F))],
        out_specs=_row_spec(),
        out_shape=jax.ShapeDtypeStruct((_N, _F), jnp.float32),
    )(h, *aggs, w, b, fw, fb)


def kernel(x, adj, fc0_W, fc0_b, mlp0_W, mlp0_b, mlp1_W, mlp1_b, fc1_W, fc1_b):
    pad = _EPAD - _E
    nblk_tot = _EPAD // (_TB * _C)
    src_b = jnp.concatenate(
        [adj[0], jnp.zeros((pad,), jnp.int32)]).reshape(nblk_tot, _TB, _C)
    dst_b = jnp.concatenate(
        [adj[1], jnp.full((pad,), _N, jnp.int32)]).reshape(nblk_tot, _TB, _C)

    fc0_b2 = fc0_b.reshape(1, _F)
    mlp0_b2 = mlp0_b.reshape(1, _F)
    mlp1_b2 = mlp1_b.reshape(1, _F)
    fw_pad = jnp.zeros((_F, _F), jnp.float32).at[:, :_NCLASS].set(fc1_W)
    fb_pad = jnp.zeros((1, _F), jnp.float32).at[0, :_NCLASS].set(fc1_b)

    h0 = _mm_relu(x, fc0_W, fc0_b2)

    agg1 = _agg(h0, src_b, dst_b)
    h1 = _gin_mm_relu(h0, [agg1[i] for i in range(_NAGG)], mlp0_W, mlp0_b2)

    agg2 = _agg(h1, src_b, dst_b)
    out = _final(h1, [agg2[i] for i in range(_NAGG)],
                 mlp1_W, mlp1_b2, fw_pad, fb_pad)

    return out[:, :_NCLASS]


# speedup vs baseline: 1.5205x; 1.5205x over previous
import functools

import jax
import jax.numpy as jnp
from jax import lax
from jax.experimental import pallas as pl
from jax.experimental.pallas import tpu as pltpu
from jax.experimental.pallas import tpu_sc as plsc

_N = 10000
_E = 320000
_F = 128
_NCLASS = 40

_NC = 2
_NS = 16
_NW = _NC * _NS
_C = 64
_TB = 16
_K = 4
_T0 = 304
_T1 = 16
_EPAD = _NS * (_T0 + _T1) * _C
_RPT = 632
_NACC = _RPT * _NS

_BR = 1000


def _make_agg_kernel():
    mesh = plsc.VectorSubcoreMesh(core_axis_name="c", subcore_axis_name="s")

    @functools.partial(
        pl.kernel,
        out_type=jax.ShapeDtypeStruct((_NC, _NACC, _F), jnp.float32),
        mesh=mesh,
        scratch_types=[
            pltpu.VMEM((_TB, _C), jnp.int32),
            pltpu.VMEM((_TB, _C), jnp.int32),
            [pltpu.VMEM((_C, _F), jnp.float32) for _ in range(_K)],
            pltpu.VMEM_SHARED((_NACC, _F), jnp.float32),
            [pltpu.SemaphoreType.DMA for _ in range(_K)],
            [pltpu.SemaphoreType.DMA for _ in range(_K)],
        ],
    )
    def agg(h_hbm, srcb_hbm, dstb_hbm,
            out_hbm, src_v, dst_v, bufs, acc_sh, gs, ps):
        c = lax.axis_index("c")
        s = lax.axis_index("s")

        @pl.when((c == 0) | (_T1 > 0))
        def _():
            zvec = jnp.zeros((16,), jnp.float32)

            def zrow(r, carry):
                for j2 in range(_F // 16):
                    bufs[0][r, pl.ds(j2 * 16, 16)] = zvec
                return carry

            lax.fori_loop(0, _C, zrow, 0)

            nfull = _RPT // _C
            rem = _RPT - nfull * _C

            def crow(k, carry):
                pltpu.sync_copy(bufs[0],
                                acc_sh.at[pl.ds(s * _RPT + k * _C, _C)])
                return carry

            lax.fori_loop(0, nfull, crow, 0)
            if rem:
                pltpu.sync_copy(
                    bufs[0].at[pl.ds(0, rem)],
                    acc_sh.at[pl.ds(s * _RPT + nfull * _C, rem)])
        plsc.subcore_barrier()

        def start_gather(t, j):
            pltpu.async_copy(h_hbm.at[src_v.at[t]], bufs[j], gs[j])

        def wait_gather(j):
            pltpu.make_async_copy(h_hbm.at[src_v.at[0]], bufs[j],
                                  gs[j]).wait()

        def start_scatter(t, j):
            pltpu.async_copy(bufs[j], acc_sh.at[dst_v.at[t]], ps[j],
                             add=True)

        def wait_scatter(j):
            pltpu.make_async_copy(bufs[j], acc_sh.at[dst_v.at[0]],
                                  ps[j]).wait()

        nb0 = _T0 // _TB
        nb1 = _T1 // _TB
        nblk = jnp.where(c == 0, nb0, nb1)
        bbase = jnp.where(c == 0, s * nb0, _NS * nb0 + s * nb1)

        def blk_body(blk, carry):
            pltpu.sync_copy(srcb_hbm.at[bbase + blk], src_v)
            pltpu.sync_copy(dstb_hbm.at[bbase + blk], dst_v)
            for j in range(_K - 1):
                start_gather(j, j)

            for j in range(_K):
                wait_gather(j)
                start_scatter(j, j)
                jn = (j + _K - 1) % _K
                if j > 0:
                    wait_scatter(jn)
                start_gather(j + _K - 1, jn)

            def group_body(g, carry2):
                base = g * _K
                for j in range(_K):
                    t = base + j
                    wait_gather(j)
                    start_scatter(t, j)
                    jn = (j + _K - 1) % _K
                    wait_scatter(jn)
                    start_gather(t + _K - 1, jn)
                return carry2

            lax.fori_loop(1, _TB // _K - 1, group_body, 0)

            base = _TB - _K
            wait_gather(0)
            start_scatter(base, 0)
            wait_scatter(_K - 1)
            start_gather(_TB - 1, _K - 1)
            for j in range(1, _K):
                wait_gather(j)
                start_scatter(base + j, j)
            for j in range(_K):
                wait_scatter(j)
            return carry

        lax.fori_loop(0, nblk, blk_body, 0)

        plsc.subcore_barrier()

        @pl.when((c == 0) | (_T1 > 0))
        def _():
            pltpu.sync_copy(acc_sh.at[pl.ds(s * _RPT, _RPT)],
                            out_hbm.at[c].at[pl.ds(s * _RPT, _RPT)])

    return agg


_agg = _make_agg_kernel()


def _mm_relu_body(x_ref, w_ref, b_ref, o_ref):
    z = jnp.dot(x_ref[...], w_ref[...], preferred_element_type=jnp.float32)
    o_ref[...] = jnp.maximum(z + b_ref[...], 0.0)


_NAGG = 2 if _T1 > 0 else 1


def _gin_mm_relu_body(h_ref, *refs):
    aggs, (w_ref, b_ref, o_ref) = refs[:_NAGG], refs[_NAGG:]
    z = h_ref[...]
    for a_ref in aggs:
        z = z + a_ref[...]
    z = jnp.dot(z, w_ref[...], preferred_element_type=jnp.float32)
    o_ref[...] = jnp.maximum(z + b_ref[...], 0.0)


def _final_body(h_ref, *refs):
    aggs, (w_ref, b_ref, fw_ref, fb_ref, o_ref) = refs[:_NAGG], refs[_NAGG:]
    z = h_ref[...]
    for a_ref in aggs:
        z = z + a_ref[...]
    z = jnp.dot(z, w_ref[...], preferred_element_type=jnp.float32)
    h2 = jnp.maximum(z + b_ref[...], 0.0)
    logits = jnp.dot(h2, fw_ref[...], preferred_element_type=jnp.float32)
    logits = logits + fb_ref[...]
    col = lax.broadcasted_iota(jnp.int32, logits.shape, 1)
    logits = jnp.where(col < _NCLASS, logits, -1e30)
    m = jnp.max(logits, axis=1, keepdims=True)
    lse = jnp.log(jnp.sum(jnp.exp(logits - m), axis=1, keepdims=True)) + m
    o_ref[...] = logits - lse


def _row_spec():
    return pl.BlockSpec((_BR, _F), lambda i: (i, 0))


def _full_spec(shape):
    return pl.BlockSpec(shape, lambda i: (0,) * len(shape))


def _grid():
    return (_N + _BR - 1) // _BR


def _mm_relu(x, w, b):
    return pl.pallas_call(
        _mm_relu_body,
        grid=(_grid(),),
        in_specs=[_row_spec(), _full_spec((_F, _F)), _full_spec((1, _F))],
        out_specs=_row_spec(),
        out_shape=jax.ShapeDtypeStruct((_N, _F), jnp.float32),
    )(x, w, b)


def _gin_mm_relu(h, aggs, w, b):
    return pl.pallas_call(
        _gin_mm_relu_body,
        grid=(_grid(),),
        in_specs=[_row_spec()] * (1 + _NAGG)
        + [_full_spec((_F, _F)), _full_spec((1, _F))],
        out_specs=_row_spec(),
        out_shape=jax.ShapeDtypeStruct((_N, _F), jnp.float32),
    )(h, *aggs, w, b)


def _final(h, aggs, w, b, fw, fb):
    return pl.pallas_call(
        _final_body,
        grid=(_grid(),),
        in_specs=[_row_spec()] * (1 + _NAGG)
        + [_full_spec((_F, _F)), _full_spec((1, _F)),
           _full_spec((_F, _F)), _full_spec((1, _F))],
        out_specs=_row_spec(),
        out_shape=jax.ShapeDtypeStruct((_N, _F), jnp.float32),
    )(h, *aggs, w, b, fw, fb)


def kernel(x, adj, fc0_W, fc0_b, mlp0_W, mlp0_b, mlp1_W, mlp1_b, fc1_W, fc1_b):
    pad = _EPAD - _E
    nblk_tot = _EPAD // (_TB * _C)
    src_b = jnp.concatenate(
        [adj[0], jnp.zeros((pad,), jnp.int32)]).reshape(nblk_tot, _TB, _C)
    dst_b = jnp.concatenate(
        [adj[1], jnp.full((pad,), _N, jnp.int32)]).reshape(nblk_tot, _TB, _C)

    fc0_b2 = fc0_b.reshape(1, _F)
    mlp0_b2 = mlp0_b.reshape(1, _F)
    mlp1_b2 = mlp1_b.reshape(1, _F)
    fw_pad = jnp.zeros((_F, _F), jnp.float32).at[:, :_NCLASS].set(fc1_W)
    fb_pad = jnp.zeros((1, _F), jnp.float32).at[0, :_NCLASS].set(fc1_b)

    h0 = _mm_relu(x, fc0_W, fc0_b2)

    agg1 = _agg(h0, src_b, dst_b)
    h1 = _gin_mm_relu(h0, [agg1[i] for i in range(_NAGG)], mlp0_W, mlp0_b2)

    agg2 = _agg(h1, src_b, dst_b)
    out = _final(h1, [agg2[i] for i in range(_NAGG)],
                 mlp1_W, mlp1_b2, fw_pad, fb_pad)

    return out[:, :_NCLASS]
